# speedup vs baseline: 6.8812x; 6.8812x over previous
import functools

import jax
import jax.numpy as jnp
from jax import lax
from jax.experimental import pallas as pl
from jax.experimental.pallas import tpu as pltpu
from jax.experimental.pallas import tpu_sc as plsc

NUM_EMBEDDINGS = 100000
EMBEDDING_DIM = 128
BATCH = 4096
HIST_LEN = 200

_INFO = plsc.get_sparse_core_info()
NC, NS, L = _INFO.num_cores, _INFO.num_subcores, _INFO.num_lanes
NW = NC * NS

TOTAL = BATCH * HIST_LEN
IDX_COLS = 128
IDX_ROWS = TOTAL // IDX_COLS
ROWS_PER_W = IDX_ROWS // NW
G = 2
CHUNK = G * IDX_COLS
NITER = ROWS_PER_W // G


def _body(idx_hbm, table_hbm, out_hbm, idx_v, rows_v, sem):
    c = lax.axis_index("c")
    s = lax.axis_index("s")
    wid = s * NC + c

    def step(it, _):
        rbase = wid * ROWS_PER_W + it * G
        pltpu.sync_copy(idx_hbm.at[pl.ds(rbase, G)], idx_v)
        copies = []
        for j in range(G):
            copies.append(
                pltpu.async_copy(
                    table_hbm.at[idx_v.at[j]],
                    rows_v.at[pl.ds(j * IDX_COLS, IDX_COLS)],
                    sem,
                )
            )
        for cp in copies:
            cp.wait()
        pltpu.sync_copy(rows_v, out_hbm.at[pl.ds(rbase * IDX_COLS, CHUNK)])
        return 0

    lax.fori_loop(0, NITER, step, 0)


@jax.jit
def _embed(input_flat2d, weight):
    kern = pl.kernel(
        _body,
        out_type=jax.ShapeDtypeStruct((TOTAL, EMBEDDING_DIM), jnp.float32),
        mesh=plsc.VectorSubcoreMesh(core_axis_name="c", subcore_axis_name="s"),
        scratch_types=[
            pltpu.VMEM((G, IDX_COLS), jnp.int32),
            pltpu.VMEM((CHUNK, EMBEDDING_DIM), jnp.float32),
            pltpu.SemaphoreType.DMA,
        ],
    )
    return kern(input_flat2d, weight)


def kernel(input_, weight):
    idx2d = input_.reshape(IDX_ROWS, IDX_COLS).astype(jnp.int32)
    out = _embed(idx2d, weight)
    return out.reshape(BATCH, HIST_LEN, EMBEDDING_DIM)


# speedup vs baseline: 9.1838x; 1.3346x over previous
import jax
import jax.numpy as jnp
from jax import lax
from jax.experimental import pallas as pl
from jax.experimental.pallas import tpu as pltpu
from jax.experimental.pallas import tpu_sc as plsc

NUM_EMBEDDINGS = 100000
EMBEDDING_DIM = 128
BATCH = 4096
HIST_LEN = 200

_INFO = plsc.get_sparse_core_info()
NC, NS, L = _INFO.num_cores, _INFO.num_subcores, _INFO.num_lanes
NW = NC * NS

TOTAL = BATCH * HIST_LEN
IDX_COLS = 128
IDX_ROWS = TOTAL // IDX_COLS
ROWS_PER_W = IDX_ROWS // NW
G = 2
CHUNK = G * IDX_COLS
NITER = ROWS_PER_W // G


def _body(idx_hbm, table_hbm, out_hbm, idx_v, rows0, rows1,
          gsem0, gsem1, wsem0, wsem1):
    c = lax.axis_index("c")
    s = lax.axis_index("s")
    wid = s * NC + c
    rbase = wid * ROWS_PER_W

    pltpu.sync_copy(idx_hbm.at[pl.ds(rbase, ROWS_PER_W)], idx_v)

    rows = (rows0, rows1)
    gsem = (gsem0, gsem1)
    wsem = (wsem0, wsem1)

    def fire_gather(it, b):
        for j in range(G):
            pltpu.async_copy(
                table_hbm.at[idx_v.at[it * G + j]],
                rows[b].at[pl.ds(j * IDX_COLS, IDX_COLS)],
                gsem[b],
            )

    def wait_gather(b):
        pltpu.make_async_copy(table_hbm.at[pl.ds(0, CHUNK)], rows[b], gsem[b]).wait()

    def fire_write(it, b):
        pltpu.async_copy(
            rows[b],
            out_hbm.at[pl.ds((rbase + it * G) * IDX_COLS, CHUNK)],
            wsem[b],
        )

    def wait_write(b):
        pltpu.make_async_copy(table_hbm.at[pl.ds(0, CHUNK)], rows[b], wsem[b]).wait()

    fire_gather(0, 0)
    fire_gather(1, 1)
    wait_gather(0)
    fire_write(0, 0)

    def step(k, _):
        it1 = 2 * k + 1
        wait_write(0)
        fire_gather(it1 + 1, 0)
        wait_gather(1)
        fire_write(it1, 1)
        it2 = 2 * k + 2
        wait_write(1)
        fire_gather(it2 + 1, 1)
        wait_gather(0)
        fire_write(it2, 0)
        return 0

    lax.fori_loop(0, (NITER - 2) // 2, step, 0)

    wait_write(0)
    wait_gather(1)
    fire_write(NITER - 1, 1)
    wait_write(1)


@jax.jit
def _embed(input_flat2d, weight):
    kern = pl.kernel(
        _body,
        out_type=jax.ShapeDtypeStruct((TOTAL, EMBEDDING_DIM), jnp.float32),
        mesh=plsc.VectorSubcoreMesh(core_axis_name="c", subcore_axis_name="s"),
        scratch_types=[
            pltpu.VMEM((ROWS_PER_W, IDX_COLS), jnp.int32),
            pltpu.VMEM((CHUNK, EMBEDDING_DIM), jnp.float32),
            pltpu.VMEM((CHUNK, EMBEDDING_DIM), jnp.float32),
            pltpu.SemaphoreType.DMA,
            pltpu.SemaphoreType.DMA,
            pltpu.SemaphoreType.DMA,
            pltpu.SemaphoreType.DMA,
        ],
    )
    return kern(input_flat2d, weight)


def kernel(input_, weight):
    idx2d = input_.reshape(IDX_ROWS, IDX_COLS).astype(jnp.int32)
    out = _embed(idx2d, weight)
    return out.reshape(BATCH, HIST_LEN, EMBEDDING_DIM)
